# baseline (speedup 1.0000x reference)
import dataclasses
import functools

import jax
import jax.numpy as jnp
from jax.experimental import pallas as pl
from jax.experimental.pallas import tpu as pltpu
from jax.experimental.pallas import tpu_sc as plsc

B = 16
NUM_JOINTS = 8
H = 128
W = 128
K = 100
THRESH = 0.1
NROWS = B + B * NUM_JOINTS
NCAND = H * W
NEG = float("-inf")
BIGI = 2**30
CAP = 256
NCHUNKG = H // 16
ONE_BITS = 0x3F800000
N_BISECT = 16


def _nms_body(x_ref, o_ref, t_ref, f_ref):
    x = jax.nn.sigmoid(x_ref[...])
    neg = jnp.full_like(x[:, :1, :], NEG)
    m = jnp.maximum(x, jnp.concatenate([x[:, 1:, :], neg], axis=1))
    m = jnp.maximum(m, jnp.concatenate([neg, x[:, :-1, :]], axis=1))
    negc = jnp.full_like(m[:, :, :1], NEG)
    m2 = jnp.maximum(m, jnp.concatenate([m[:, :, 1:], negc], axis=2))
    m2 = jnp.maximum(m2, jnp.concatenate([negc, m[:, :, :-1]], axis=2))
    keep = (m2 == x).astype(x.dtype)
    s = x * keep
    blk = s.shape[0]

    o_ref[...] = jnp.transpose(s, (0, 2, 1))
    cm = jnp.max(s.reshape(blk, NCHUNKG, 16, W), axis=2)

    def bis(_, ab):
        a, b = ab
        mid = (a + b) // 2
        v = jax.lax.bitcast_convert_type(mid, jnp.float32)
        cnt = jnp.sum((s >= v).astype(jnp.int32), axis=(1, 2), keepdims=True)
        ge = cnt >= K
        return jnp.where(ge, mid, a), jnp.where(ge, b, mid)

    a0 = jnp.zeros((blk, 1, 1), jnp.int32)
    b0 = jnp.full((blk, 1, 1), ONE_BITS, jnp.int32)
    a, _ = jax.lax.fori_loop(0, N_BISECT, bis, (a0, b0))
    t = jax.lax.bitcast_convert_type(a, jnp.float32)
    t_ref[...] = jnp.broadcast_to(t.reshape(blk, 1), (blk, 16))
    f_ref[...] = (cm >= t).astype(jnp.int32)


def _nms_pallas(x):
    n = x.shape[0]
    blk = 8
    return pl.pallas_call(
        _nms_body,
        grid=(n // blk,),
        in_specs=[pl.BlockSpec((blk, H, W), lambda i: (i, 0, 0))],
        out_specs=[pl.BlockSpec((blk, W, H), lambda i: (i, 0, 0)),
                   pl.BlockSpec((blk, 16), lambda i: (i, 0)),
                   pl.BlockSpec((blk, NCHUNKG, W), lambda i: (i, 0, 0))],
        out_shape=[jax.ShapeDtypeStruct((n, W, H), jnp.float32),
                   jax.ShapeDtypeStruct((n, 16), jnp.float32),
                   jax.ShapeDtypeStruct((n, NCHUNKG, W), jnp.int32)],
    )(x)


def _compact_sc(x, t, f):
    mesh = plsc.VectorSubcoreMesh(core_axis_name="c", subcore_axis_name="s")
    nunits = 32
    nwaves = (NROWS + nunits - 1) // nunits

    cp = pltpu.CompilerParams()
    if "needs_layout_passes" in pltpu.CompilerParams.__dataclass_fields__:
        cp = dataclasses.replace(cp, needs_layout_passes=False)

    @functools.partial(
        pl.kernel,
        out_type=[jax.ShapeDtypeStruct((NROWS, CAP), jnp.float32),
                  jax.ShapeDtypeStruct((NROWS, CAP), jnp.int32)],
        mesh=mesh,
        compiler_params=cp,
        scratch_types=[pltpu.VMEM((NCAND,), jnp.float32),
                       pltpu.VMEM((NCAND,), jnp.float32),
                       pltpu.VMEM((16,), jnp.float32),
                       pltpu.VMEM((CAP + 16,), jnp.float32),
                       pltpu.VMEM((CAP + 16,), jnp.int32),
                       pltpu.VMEM((NCHUNKG, W), jnp.int32),
                       pltpu.SMEM((1,), jnp.int32),
                       pltpu.SemaphoreType.DMA,
                       pltpu.SemaphoreType.DMA],
    )
    def kern(x_hbm, t_hbm, f_hbm, ov_hbm, oi_hbm, xbuf0, xbuf1, tbuf, vbuf,
             ibuf, fbuf, cur, sem0, sem1):
        cid = jax.lax.axis_index("c")
        sid = jax.lax.axis_index("s")
        unit = cid * 16 + sid
        lane = jax.lax.iota(jnp.int32, 16)
        bufs = [xbuf0, xbuf1]
        sems = [sem0, sem1]

        pltpu.make_async_copy(x_hbm.at[unit], xbuf0, sem0).start()

        for w in range(nwaves):
            row = unit + w * nunits
            xbuf = bufs[w % 2]

            @pl.when(row < NROWS)
            def _(row=row, xbuf=xbuf, w=w):
                if w + 1 < nwaves:
                    nrow = row + nunits

                    @pl.when(nrow < NROWS)
                    def _(nrow=nrow, w=w):
                        pltpu.make_async_copy(
                            x_hbm.at[nrow], bufs[(w + 1) % 2],
                            sems[(w + 1) % 2]).start()

                pltpu.sync_copy(f_hbm.at[row], fbuf)
                pltpu.sync_copy(t_hbm.at[row], tbuf)
                tval = tbuf[...][0]
                cur[0] = 0

                @pl.loop(0, (CAP + 16) // 16)
                def _p(p):
                    vbuf[pl.ds(p * 16, 16)] = jnp.full((16,), NEG,
                                                       jnp.float32)
                    ibuf[pl.ds(p * 16, 16)] = jnp.full((16,), BIGI,
                                                       jnp.int32)

                pltpu.make_async_copy(x_hbm.at[row], xbuf,
                                      sems[w % 2]).wait()

                @pl.loop(0, NCHUNKG)
                def _g(g):
                    @pl.loop(0, W // 16)
                    def _xg(xg):
                        fvec = fbuf[g, pl.ds(xg * 16, 16)]

                        @pl.when(jnp.max(fvec) != 0)
                        def _():
                            for l in range(16):
                                @pl.when(fvec[l] != 0)
                                def _(l=l):
                                    xc = xg * 16 + l
                                    c = xc * NCHUNKG + g
                                    vec = xbuf[pl.ds(c * 16, 16)]
                                    msk = vec >= tval
                                    ix = c * 16 + lane
                                    orig = ((ix & (H - 1)) << 7) + (ix >> 7)
                                    keys = jnp.where(msk, orig, BIGI)
                                    sk, sv = plsc.sort_key_val(keys, vec)
                                    svv = jnp.where(sk < BIGI, sv, NEG)
                                    cc = cur[0]

                                    @pl.when(cc <= CAP)
                                    def _():
                                        vbuf[pl.ds(cc, 16)] = svv
                                        ibuf[pl.ds(cc, 16)] = sk
                                    cur[0] = cc + jnp.sum(
                                        msk.astype(jnp.int32))

                pltpu.sync_copy(vbuf.at[pl.ds(0, CAP)], ov_hbm.at[row])
                pltpu.sync_copy(ibuf.at[pl.ds(0, CAP)], oi_hbm.at[row])

    return kern(x, t, f)


def _sort_body(v_ref, i_ref, vals_ref, inds_ref):
    sv0 = v_ref[...]
    si = i_ref[...]
    koto = jax.lax.broadcasted_iota(jnp.int32, (NROWS, 128), 1)

    def body(j, carry):
        sv, vacc, iacc = carry
        m = jnp.max(sv, axis=1, keepdims=True)
        idx = jnp.min(jnp.where(sv == m, si, BIGI), axis=1, keepdims=True)
        sv = jnp.where((sv == m) & (si == idx), NEG, sv)
        vacc = jnp.where(koto == j, m, vacc)
        iacc = jnp.where(koto == j, idx, iacc)
        return sv, vacc, iacc

    _, vacc, iacc = jax.lax.fori_loop(
        0, K, body,
        (sv0, jnp.zeros((NROWS, 128), jnp.float32),
         jnp.zeros((NROWS, 128), jnp.int32)))
    vals_ref[...] = vacc
    inds_ref[...] = iacc


def _sort_pallas(sval, sidx):
    return pl.pallas_call(
        _sort_body,
        out_shape=[jax.ShapeDtypeStruct((NROWS, 128), jnp.float32),
                   jax.ShapeDtypeStruct((NROWS, 128), jnp.int32)],
    )(sval, sidx)


def _decode_body(f_ref, hsc_ref, hx_ref, hy_ref, bb_ref, kf_ref, kd_ref):
    f = f_ref[0]
    hps = f[:, 0:16]
    regx = f[:, 16:17]
    regy = f[:, 17:18]
    whw = f[:, 18:19]
    whh = f[:, 19:20]
    xsi = f[:, 20:21]
    ysi = f[:, 21:22]
    xs = xsi + regx
    ys = ysi + regy
    l = xs - whw / 2
    t = ys - whh / 2
    r = xs + whw / 2
    btm = ys + whh / 2
    bbox = jnp.concatenate([l, t, r, btm], axis=1)
    bb_ref[0] = bbox[:K]

    ii = jax.lax.broadcasted_iota(jnp.int32, (128, 16), 1)
    kd = hps + jnp.where(ii % 2 == 0, xsi, ysi)
    kd_ref[0] = kd[:K]

    klane = jax.lax.broadcasted_iota(jnp.int32, (128, 128), 1)
    kf = jnp.zeros((128, 16), jnp.float32)
    for j in range(NUM_JOINTS):
        kx = hps[:, 2 * j:2 * j + 1] + xsi
        ky = hps[:, 2 * j + 1:2 * j + 2] + ysi
        hsc = hsc_ref[0, j:j + 1, :]
        msk = hsc > THRESH
        hscm = jnp.where(msk, hsc, -1.0)
        hx = jnp.where(msk, hx_ref[0, j:j + 1, :], -10000.0)
        hy = jnp.where(msk, hy_ref[0, j:j + 1, :], -10000.0)
        dx = kx - hx
        dy = ky - hy
        dist = jnp.sqrt(dx * dx + dy * dy)
        dist = jnp.where(klane < K, dist, jnp.inf)
        mind = jnp.min(dist, axis=1, keepdims=True)
        mini = jnp.min(jnp.where(dist == mind, klane, BIGI), axis=1,
                       keepdims=True)
        selm = klane == mini
        hsg = jnp.sum(jnp.where(selm, jnp.broadcast_to(hscm, (128, 128)), 0.0),
                      axis=1, keepdims=True)
        hxg = jnp.sum(jnp.where(selm, jnp.broadcast_to(hx, (128, 128)), 0.0),
                      axis=1, keepdims=True)
        hyg = jnp.sum(jnp.where(selm, jnp.broadcast_to(hy, (128, 128)), 0.0),
                      axis=1, keepdims=True)
        m6 = ((hxg < l).astype(jnp.int32) + (hxg > r).astype(jnp.int32)
              + (hyg < t).astype(jnp.int32) + (hyg > btm).astype(jnp.int32)
              + (hsg < THRESH).astype(jnp.int32)
              + (mind > jnp.maximum(btm - t, r - l) * 0.3).astype(jnp.int32))
        maskf = m6 > 0
        kfx = jnp.where(maskf, kx, hxg)
        kfy = jnp.where(maskf, ky, hyg)
        kf = jnp.where(ii == 2 * j, kfx, kf)
        kf = jnp.where(ii == 2 * j + 1, kfy, kf)
    kf_ref[0] = kf[:K]


def _decode_pallas(feats, hsc, hxr, hyr):
    return pl.pallas_call(
        _decode_body,
        grid=(B,),
        in_specs=[pl.BlockSpec((1, 128, 23), lambda i: (i, 0, 0)),
                  pl.BlockSpec((1, NUM_JOINTS, 128), lambda i: (i, 0, 0)),
                  pl.BlockSpec((1, NUM_JOINTS, 128), lambda i: (i, 0, 0)),
                  pl.BlockSpec((1, NUM_JOINTS, 128), lambda i: (i, 0, 0))],
        out_specs=[pl.BlockSpec((1, K, 4), lambda i: (i, 0, 0)),
                   pl.BlockSpec((1, K, 16), lambda i: (i, 0, 0)),
                   pl.BlockSpec((1, K, 16), lambda i: (i, 0, 0))],
        out_shape=[jax.ShapeDtypeStruct((B, K, 4), jnp.float32),
                   jax.ShapeDtypeStruct((B, K, 16), jnp.float32),
                   jax.ShapeDtypeStruct((B, K, 16), jnp.float32)],
    )(feats, hsc, hxr, hyr)


def _gather_feat(feat, ind):
    b, n, c = feat.shape
    k = ind.shape[1]
    ind_b = jnp.broadcast_to(ind[:, :, None], (b, k, c))
    return jnp.take_along_axis(feat, ind_b, axis=1)


def _tg(feat, ind):
    b, c, h, w = feat.shape
    k = ind.shape[1]
    f2 = feat.reshape(b, c, h * w)
    g = jnp.take_along_axis(
        f2, jnp.broadcast_to(ind[:, None, :], (b, c, k)), axis=2)
    return jnp.transpose(g, (0, 2, 1))


def kernel(hm, wh, hps, reg, hm_hp, hp_offset, scale):
    batch = B
    num_joints = NUM_JOINTS

    rows = jnp.concatenate(
        [hm.reshape(B, H, W), hm_hp.reshape(B * NUM_JOINTS, H, W)], axis=0)
    nmsd_t, tvec, flags = _nms_pallas(rows)
    sval, sidx = _compact_sc(nmsd_t.reshape(NROWS, NCAND), tvec, flags)
    vals, inds_all = _sort_pallas(sval, sidx)

    scores = vals[:B, :K].reshape(B, K, 1)
    inds = inds_all[:B, :K]
    clses = jnp.zeros((B, K, 1), jnp.float32)

    hm_inds = inds_all[B:, :K].reshape(B, NUM_JOINTS, K)

    hps_g = _tg(hps, inds)
    reg_g = _tg(reg, inds)
    wh_g = _tg(wh, inds)
    obj_scale = _tg(scale, inds).reshape(batch, K, 3)
    hp_off = _tg(hp_offset, hm_inds.reshape(batch, num_joints * K)).reshape(
        batch, num_joints, K, 2)

    hm_x_raw = (hm_inds % W).astype(jnp.float32) + hp_off[:, :, :, 0]
    hm_y_raw = (hm_inds // W).astype(jnp.float32) + hp_off[:, :, :, 1]
    pad_k = [(0, 0), (0, 0), (0, 128 - K)]
    hsc = jnp.pad(vals[B:].reshape(B, NUM_JOINTS, 128)[:, :, :K], pad_k)
    hxr = jnp.pad(hm_x_raw, pad_k)
    hyr = jnp.pad(hm_y_raw, pad_k)

    xs_i = (inds_all[:B] % W).astype(jnp.float32)[:, :, None]
    ys_i = (inds_all[:B] // W).astype(jnp.float32)[:, :, None]
    padr = [(0, 0), (0, 128 - K), (0, 0)]
    feats = jnp.concatenate(
        [jnp.pad(hps_g, padr), jnp.pad(reg_g, padr), jnp.pad(wh_g, padr),
         xs_i, ys_i], axis=2)
    feats = jnp.pad(feats, [(0, 0), (0, 0), (0, 1)])

    bboxes, kps_final, kps_displacement_mean = _decode_pallas(
        feats, hsc, hxr, hyr)
    kps_heatmap_mean = jnp.full((B, K, NUM_JOINTS * 2), -10000.0, jnp.float32)
    return (bboxes, scores, kps_final, clses, obj_scale,
            kps_displacement_mean, kps_heatmap_mean)


# speedup vs baseline: 1.0125x; 1.0125x over previous
import dataclasses
import functools

import jax
import jax.numpy as jnp
from jax.experimental import pallas as pl
from jax.experimental.pallas import tpu as pltpu
from jax.experimental.pallas import tpu_sc as plsc

B = 16
NUM_JOINTS = 8
H = 128
W = 128
K = 100
THRESH = 0.1
NROWS = B + B * NUM_JOINTS
NCAND = H * W
NEG = float("-inf")
BIGI = 2**30
CAP = 256
NCHUNKG = H // 16
ONE_BITS = 0x3F800000
N_BISECT = 16


def _nms_body(x_ref, o_ref, t_ref, f_ref):
    x = jax.nn.sigmoid(x_ref[...])
    neg = jnp.full_like(x[:, :1, :], NEG)
    m = jnp.maximum(x, jnp.concatenate([x[:, 1:, :], neg], axis=1))
    m = jnp.maximum(m, jnp.concatenate([neg, x[:, :-1, :]], axis=1))
    negc = jnp.full_like(m[:, :, :1], NEG)
    m2 = jnp.maximum(m, jnp.concatenate([m[:, :, 1:], negc], axis=2))
    m2 = jnp.maximum(m2, jnp.concatenate([negc, m[:, :, :-1]], axis=2))
    keep = (m2 == x).astype(x.dtype)
    s = x * keep
    blk = s.shape[0]

    o_ref[...] = jnp.transpose(s, (0, 2, 1))
    cm = jnp.max(s.reshape(blk, NCHUNKG, 16, W), axis=2)

    def bis(_, ab):
        a, b = ab
        mid = (a + b) // 2
        v = jax.lax.bitcast_convert_type(mid, jnp.float32)
        cnt = jnp.sum((s >= v).astype(jnp.int32), axis=(1, 2), keepdims=True)
        ge = cnt >= K
        return jnp.where(ge, mid, a), jnp.where(ge, b, mid)

    a0 = jnp.zeros((blk, 1, 1), jnp.int32)
    b0 = jnp.full((blk, 1, 1), ONE_BITS, jnp.int32)
    a, _ = jax.lax.fori_loop(0, N_BISECT, bis, (a0, b0))
    t = jax.lax.bitcast_convert_type(a, jnp.float32)
    t_ref[...] = jnp.broadcast_to(t.reshape(blk, 1), (blk, 16))
    f_ref[...] = (cm >= t).astype(jnp.int32)


def _nms_pallas(x):
    n = x.shape[0]
    blk = 8
    return pl.pallas_call(
        _nms_body,
        grid=(n // blk,),
        in_specs=[pl.BlockSpec((blk, H, W), lambda i: (i, 0, 0))],
        out_specs=[pl.BlockSpec((blk, W, H), lambda i: (i, 0, 0)),
                   pl.BlockSpec((blk, 16), lambda i: (i, 0)),
                   pl.BlockSpec((blk, NCHUNKG, W), lambda i: (i, 0, 0))],
        out_shape=[jax.ShapeDtypeStruct((n, W, H), jnp.float32),
                   jax.ShapeDtypeStruct((n, 16), jnp.float32),
                   jax.ShapeDtypeStruct((n, NCHUNKG, W), jnp.int32)],
    )(x)


def _compact_sc(x, t, f):
    mesh = plsc.VectorSubcoreMesh(core_axis_name="c", subcore_axis_name="s")
    nunits = 32
    nwaves = (NROWS + nunits - 1) // nunits

    cp = pltpu.CompilerParams()
    if "needs_layout_passes" in pltpu.CompilerParams.__dataclass_fields__:
        cp = dataclasses.replace(cp, needs_layout_passes=False)

    @functools.partial(
        pl.kernel,
        out_type=[jax.ShapeDtypeStruct((NROWS, CAP), jnp.float32),
                  jax.ShapeDtypeStruct((NROWS, CAP), jnp.int32)],
        mesh=mesh,
        compiler_params=cp,
        scratch_types=[pltpu.VMEM((NCAND,), jnp.float32),
                       pltpu.VMEM((16,), jnp.float32),
                       pltpu.VMEM((CAP + 16,), jnp.float32),
                       pltpu.VMEM((CAP + 16,), jnp.int32),
                       pltpu.VMEM((NCHUNKG, W), jnp.int32),
                       pltpu.SMEM((1,), jnp.int32),
                       pltpu.SemaphoreType.DMA],
    )
    def kern(x_hbm, t_hbm, f_hbm, ov_hbm, oi_hbm, xbuf, tbuf, vbuf,
             ibuf, fbuf, cur, sem):
        cid = jax.lax.axis_index("c")
        sid = jax.lax.axis_index("s")
        unit = cid * 16 + sid
        lane = jax.lax.iota(jnp.int32, 16)

        @pl.loop(0, nwaves)
        def _w(w):
            row = unit + w * nunits

            @pl.when(row < NROWS)
            def _():
                xcopy = pltpu.make_async_copy(x_hbm.at[row], xbuf, sem)
                xcopy.start()
                pltpu.sync_copy(f_hbm.at[row], fbuf)
                pltpu.sync_copy(t_hbm.at[row], tbuf)
                tval = tbuf[...][0]
                cur[0] = 0

                @pl.loop(0, (CAP + 16) // 16)
                def _p(p):
                    vbuf[pl.ds(p * 16, 16)] = jnp.full((16,), NEG,
                                                       jnp.float32)
                    ibuf[pl.ds(p * 16, 16)] = jnp.full((16,), BIGI,
                                                       jnp.int32)

                xcopy.wait()

                @pl.loop(0, NCHUNKG)
                def _g(g):
                    @pl.loop(0, W // 16)
                    def _xg(xg):
                        fvec = fbuf[g, pl.ds(xg * 16, 16)]

                        @pl.when(jnp.max(fvec) != 0)
                        def _():
                            for l in range(16):
                                @pl.when(fvec[l] != 0)
                                def _(l=l):
                                    xc = xg * 16 + l
                                    c = xc * NCHUNKG + g
                                    vec = xbuf[pl.ds(c * 16, 16)]
                                    msk = vec >= tval
                                    ix = c * 16 + lane
                                    orig = ((ix & (H - 1)) << 7) + (ix >> 7)
                                    keys = jnp.where(msk, orig, BIGI)
                                    sk, sv = plsc.sort_key_val(keys, vec)
                                    svv = jnp.where(sk < BIGI, sv, NEG)
                                    cc = cur[0]

                                    @pl.when(cc <= CAP)
                                    def _():
                                        vbuf[pl.ds(cc, 16)] = svv
                                        ibuf[pl.ds(cc, 16)] = sk
                                    cur[0] = cc + jnp.sum(
                                        msk.astype(jnp.int32))

                pltpu.sync_copy(vbuf.at[pl.ds(0, CAP)], ov_hbm.at[row])
                pltpu.sync_copy(ibuf.at[pl.ds(0, CAP)], oi_hbm.at[row])

    return kern(x, t, f)


def _sort_body(v_ref, i_ref, vals_ref, inds_ref):
    sv0 = v_ref[...]
    si = i_ref[...]
    koto = jax.lax.broadcasted_iota(jnp.int32, (NROWS, 128), 1)

    def body(j, carry):
        sv, vacc, iacc = carry
        m = jnp.max(sv, axis=1, keepdims=True)
        idx = jnp.min(jnp.where(sv == m, si, BIGI), axis=1, keepdims=True)
        sv = jnp.where((sv == m) & (si == idx), NEG, sv)
        vacc = jnp.where(koto == j, m, vacc)
        iacc = jnp.where(koto == j, idx, iacc)
        return sv, vacc, iacc

    _, vacc, iacc = jax.lax.fori_loop(
        0, K, body,
        (sv0, jnp.zeros((NROWS, 128), jnp.float32),
         jnp.zeros((NROWS, 128), jnp.int32)))
    vals_ref[...] = vacc
    inds_ref[...] = iacc


def _sort_pallas(sval, sidx):
    return pl.pallas_call(
        _sort_body,
        out_shape=[jax.ShapeDtypeStruct((NROWS, 128), jnp.float32),
                   jax.ShapeDtypeStruct((NROWS, 128), jnp.int32)],
    )(sval, sidx)


def _decode_body(f_ref, hsc_ref, hx_ref, hy_ref, bb_ref, kf_ref, kd_ref):
    f = f_ref[0]
    hps = f[:, 0:16]
    regx = f[:, 16:17]
    regy = f[:, 17:18]
    whw = f[:, 18:19]
    whh = f[:, 19:20]
    xsi = f[:, 20:21]
    ysi = f[:, 21:22]
    xs = xsi + regx
    ys = ysi + regy
    l = xs - whw / 2
    t = ys - whh / 2
    r = xs + whw / 2
    btm = ys + whh / 2
    bbox = jnp.concatenate([l, t, r, btm], axis=1)
    bb_ref[0] = bbox[:K]

    ii = jax.lax.broadcasted_iota(jnp.int32, (128, 16), 1)
    kd = hps + jnp.where(ii % 2 == 0, xsi, ysi)
    kd_ref[0] = kd[:K]

    klane = jax.lax.broadcasted_iota(jnp.int32, (128, 128), 1)
    kf = jnp.zeros((128, 16), jnp.float32)
    for j in range(NUM_JOINTS):
        kx = hps[:, 2 * j:2 * j + 1] + xsi
        ky = hps[:, 2 * j + 1:2 * j + 2] + ysi
        hsc = hsc_ref[0, j:j + 1, :]
        msk = hsc > THRESH
        hscm = jnp.where(msk, hsc, -1.0)
        hx = jnp.where(msk, hx_ref[0, j:j + 1, :], -10000.0)
        hy = jnp.where(msk, hy_ref[0, j:j + 1, :], -10000.0)
        dx = kx - hx
        dy = ky - hy
        dist = jnp.sqrt(dx * dx + dy * dy)
        dist = jnp.where(klane < K, dist, jnp.inf)
        mind = jnp.min(dist, axis=1, keepdims=True)
        mini = jnp.min(jnp.where(dist == mind, klane, BIGI), axis=1,
                       keepdims=True)
        selm = klane == mini
        hsg = jnp.sum(jnp.where(selm, jnp.broadcast_to(hscm, (128, 128)), 0.0),
                      axis=1, keepdims=True)
        hxg = jnp.sum(jnp.where(selm, jnp.broadcast_to(hx, (128, 128)), 0.0),
                      axis=1, keepdims=True)
        hyg = jnp.sum(jnp.where(selm, jnp.broadcast_to(hy, (128, 128)), 0.0),
                      axis=1, keepdims=True)
        m6 = ((hxg < l).astype(jnp.int32) + (hxg > r).astype(jnp.int32)
              + (hyg < t).astype(jnp.int32) + (hyg > btm).astype(jnp.int32)
              + (hsg < THRESH).astype(jnp.int32)
              + (mind > jnp.maximum(btm - t, r - l) * 0.3).astype(jnp.int32))
        maskf = m6 > 0
        kfx = jnp.where(maskf, kx, hxg)
        kfy = jnp.where(maskf, ky, hyg)
        kf = jnp.where(ii == 2 * j, kfx, kf)
        kf = jnp.where(ii == 2 * j + 1, kfy, kf)
    kf_ref[0] = kf[:K]


def _decode_pallas(feats, hsc, hxr, hyr):
    return pl.pallas_call(
        _decode_body,
        grid=(B,),
        in_specs=[pl.BlockSpec((1, 128, 23), lambda i: (i, 0, 0)),
                  pl.BlockSpec((1, NUM_JOINTS, 128), lambda i: (i, 0, 0)),
                  pl.BlockSpec((1, NUM_JOINTS, 128), lambda i: (i, 0, 0)),
                  pl.BlockSpec((1, NUM_JOINTS, 128), lambda i: (i, 0, 0))],
        out_specs=[pl.BlockSpec((1, K, 4), lambda i: (i, 0, 0)),
                   pl.BlockSpec((1, K, 16), lambda i: (i, 0, 0)),
                   pl.BlockSpec((1, K, 16), lambda i: (i, 0, 0))],
        out_shape=[jax.ShapeDtypeStruct((B, K, 4), jnp.float32),
                   jax.ShapeDtypeStruct((B, K, 16), jnp.float32),
                   jax.ShapeDtypeStruct((B, K, 16), jnp.float32)],
    )(feats, hsc, hxr, hyr)


def _gather_feat(feat, ind):
    b, n, c = feat.shape
    k = ind.shape[1]
    ind_b = jnp.broadcast_to(ind[:, :, None], (b, k, c))
    return jnp.take_along_axis(feat, ind_b, axis=1)


def _tg(feat, ind):
    b, c, h, w = feat.shape
    k = ind.shape[1]
    f2 = feat.reshape(b, c, h * w)
    g = jnp.take_along_axis(
        f2, jnp.broadcast_to(ind[:, None, :], (b, c, k)), axis=2)
    return jnp.transpose(g, (0, 2, 1))


def kernel(hm, wh, hps, reg, hm_hp, hp_offset, scale):
    batch = B
    num_joints = NUM_JOINTS

    rows = jnp.concatenate(
        [hm.reshape(B, H, W), hm_hp.reshape(B * NUM_JOINTS, H, W)], axis=0)
    nmsd_t, tvec, flags = _nms_pallas(rows)
    sval, sidx = _compact_sc(nmsd_t.reshape(NROWS, NCAND), tvec, flags)
    vals, inds_all = _sort_pallas(sval, sidx)

    scores = vals[:B, :K].reshape(B, K, 1)
    inds = inds_all[:B, :K]
    clses = jnp.zeros((B, K, 1), jnp.float32)

    hm_inds = inds_all[B:, :K].reshape(B, NUM_JOINTS, K)

    hps_g = _tg(hps, inds)
    reg_g = _tg(reg, inds)
    wh_g = _tg(wh, inds)
    obj_scale = _tg(scale, inds).reshape(batch, K, 3)
    hp_off = _tg(hp_offset, hm_inds.reshape(batch, num_joints * K)).reshape(
        batch, num_joints, K, 2)

    hm_x_raw = (hm_inds % W).astype(jnp.float32) + hp_off[:, :, :, 0]
    hm_y_raw = (hm_inds // W).astype(jnp.float32) + hp_off[:, :, :, 1]
    pad_k = [(0, 0), (0, 0), (0, 128 - K)]
    hsc = jnp.pad(vals[B:].reshape(B, NUM_JOINTS, 128)[:, :, :K], pad_k)
    hxr = jnp.pad(hm_x_raw, pad_k)
    hyr = jnp.pad(hm_y_raw, pad_k)

    xs_i = (inds_all[:B] % W).astype(jnp.float32)[:, :, None]
    ys_i = (inds_all[:B] // W).astype(jnp.float32)[:, :, None]
    padr = [(0, 0), (0, 128 - K), (0, 0)]
    feats = jnp.concatenate(
        [jnp.pad(hps_g, padr), jnp.pad(reg_g, padr), jnp.pad(wh_g, padr),
         xs_i, ys_i], axis=2)
    feats = jnp.pad(feats, [(0, 0), (0, 0), (0, 1)])

    bboxes, kps_final, kps_displacement_mean = _decode_pallas(
        feats, hsc, hxr, hyr)
    kps_heatmap_mean = jnp.full((B, K, NUM_JOINTS * 2), -10000.0, jnp.float32)
    return (bboxes, scores, kps_final, clses, obj_scale,
            kps_displacement_mean, kps_heatmap_mean)
